# initial kernel scaffold (unmeasured)
import jax
import jax.numpy as jnp
from jax import lax
from jax.experimental import pallas as pl
from jax.experimental.pallas import tpu as pltpu

N_DEV = 16
N_EXP_LOCAL = 4


def kernel(x, router_W, route_idx, expert_W):
    n_tok, d_model = x.shape
    n_loc, _, d_ff = expert_W.shape

    def body(x_ref, rw_ref, idx_ref, ew_ref, out_ref,
             comm_ref, send_sems, recv_sems):
        my = lax.axis_index("i")
        left = lax.rem(my + N_DEV - 1, N_DEV)
        right = lax.rem(my + 1, N_DEV)

        barrier_sem = pltpu.get_barrier_semaphore()
        for nbr in (left, right):
            pl.semaphore_signal(
                barrier_sem, inc=1,
                device_id=(nbr,), device_id_type=pl.DeviceIdType.MESH,
            )
        pl.semaphore_wait(barrier_sem, 2)

        xv = x_ref[...]
        scores = jnp.dot(xv, rw_ref[...], preferred_element_type=jnp.float32)
        m = jnp.max(scores, axis=1, keepdims=True)
        p = jnp.exp(scores - m)
        probs = p / jnp.sum(p, axis=1, keepdims=True)
        e0 = idx_ref[:, 0:1]
        e1 = idx_ref[:, 1:2]
        iota = lax.broadcasted_iota(jnp.int32, probs.shape, 1)
        p0 = jnp.sum(jnp.where(iota == e0, probs, 0.0), axis=1, keepdims=True)
        p1 = jnp.sum(jnp.where(iota == e1, probs, 0.0), axis=1, keepdims=True)
        gsum = p0 + p1
        g0 = p0 / gsum
        g1 = p1 / gsum

        for h in range(N_DEV):
            send_slot = h % 2
            recv_slot = (h + 1) % 2
            src = ew_ref if h == 0 else comm_ref.at[send_slot]
            if h < N_DEV - 1:
                rdma = pltpu.make_async_remote_copy(
                    src_ref=src,
                    dst_ref=comm_ref.at[recv_slot],
                    send_sem=send_sems.at[send_slot],
                    recv_sem=recv_sems.at[recv_slot],
                    device_id=(right,),
                    device_id_type=pl.DeviceIdType.MESH,
                )
                rdma.start()

            srcdev = lax.rem(my - h + 2 * N_DEV, N_DEV)
            contrib = None
            for e in range(N_EXP_LOCAL):
                e_glob = srcdev * N_EXP_LOCAL + e
                coeff = (jnp.where(e0 == e_glob, g0, 0.0)
                         + jnp.where(e1 == e_glob, g1, 0.0))
                we = ew_ref[e] if h == 0 else comm_ref[send_slot, e]
                y = jnp.dot(xv * coeff, we,
                            preferred_element_type=jnp.float32)
                contrib = y if contrib is None else contrib + y
            if h == 0:
                out_ref[...] = contrib
            else:
                out_ref[...] = out_ref[...] + contrib

            if h < N_DEV - 1:
                rdma.wait()

    return pl.pallas_call(
        body,
        out_shape=jax.ShapeDtypeStruct((n_tok, d_ff), jnp.float32),
        in_specs=[pl.BlockSpec(memory_space=pltpu.VMEM)] * 4,
        out_specs=pl.BlockSpec(memory_space=pltpu.VMEM),
        scratch_shapes=[
            pltpu.VMEM((2, n_loc, d_model, d_ff), jnp.float32),
            pltpu.SemaphoreType.DMA((2,)),
            pltpu.SemaphoreType.DMA((2,)),
        ],
        compiler_params=pltpu.CompilerParams(collective_id=0),
    )(x, router_W, route_idx, expert_W)


# baseline (device time: 1412256 ns/iter reference)
import jax
import jax.numpy as jnp
from jax import lax
from jax.experimental import pallas as pl
from jax.experimental.pallas import tpu as pltpu

N_DEV = 16
N_EXP_LOCAL = 4


def kernel(x, router_W, route_idx, expert_W):
    n_tok, d_model = x.shape
    n_loc, _, d_ff = expert_W.shape

    def body(x_ref, rw_ref, idx_ref, ew_ref, out_ref,
             comm_ref, send_sems, recv_sems):
        my = lax.axis_index("i")
        left = lax.rem(my + N_DEV - 1, N_DEV)
        right = lax.rem(my + 1, N_DEV)

        barrier_sem = pltpu.get_barrier_semaphore()
        for nbr in (left, right):
            pl.semaphore_signal(
                barrier_sem, inc=1,
                device_id=(nbr,), device_id_type=pl.DeviceIdType.MESH,
            )
        pl.semaphore_wait(barrier_sem, 2)

        xv = x_ref[...]
        scores = jnp.dot(xv, rw_ref[...], preferred_element_type=jnp.float32)
        m = jnp.max(scores, axis=1, keepdims=True)
        p = jnp.exp(scores - m)
        probs = p / jnp.sum(p, axis=1, keepdims=True)
        e0 = idx_ref[:, 0:1]
        e1 = idx_ref[:, 1:2]
        iota = lax.broadcasted_iota(jnp.int32, probs.shape, 1)
        p0 = jnp.sum(jnp.where(iota == e0, probs, 0.0), axis=1, keepdims=True)
        p1 = jnp.sum(jnp.where(iota == e1, probs, 0.0), axis=1, keepdims=True)
        gsum = p0 + p1
        g0 = p0 / gsum
        g1 = p1 / gsum

        out_ref[...] = jnp.zeros((n_tok, d_ff), jnp.float32)
        comm_ref[0] = ew_ref[...]

        def hop(h, carry):
            send_slot = lax.rem(h, 2)
            recv_slot = lax.rem(h + 1, 2)
            rdma = pltpu.make_async_remote_copy(
                src_ref=comm_ref.at[send_slot],
                dst_ref=comm_ref.at[recv_slot],
                send_sem=send_sems.at[send_slot],
                recv_sem=recv_sems.at[recv_slot],
                device_id=(right,),
                device_id_type=pl.DeviceIdType.MESH,
            )

            @pl.when(h < N_DEV - 1)
            def _():
                rdma.start()

            srcdev = lax.rem(my - h + 2 * N_DEV, N_DEV)
            contrib = None
            for e in range(N_EXP_LOCAL):
                e_glob = srcdev * N_EXP_LOCAL + e
                coeff = (jnp.where(e0 == e_glob, g0, 0.0)
                         + jnp.where(e1 == e_glob, g1, 0.0))
                we = comm_ref[send_slot, e]
                y = jnp.dot(xv * coeff, we,
                            preferred_element_type=jnp.float32)
                contrib = y if contrib is None else contrib + y
            out_ref[...] = out_ref[...] + contrib

            @pl.when(h < N_DEV - 1)
            def _():
                rdma.wait()

            return carry

        lax.fori_loop(0, N_DEV, hop, 0)

    return pl.pallas_call(
        body,
        out_shape=jax.ShapeDtypeStruct((n_tok, d_ff), jnp.float32),
        in_specs=[pl.BlockSpec(memory_space=pltpu.VMEM)] * 4,
        out_specs=pl.BlockSpec(memory_space=pltpu.VMEM),
        scratch_shapes=[
            pltpu.VMEM((2, n_loc, d_model, d_ff), jnp.float32),
            pltpu.SemaphoreType.DMA((2,)),
            pltpu.SemaphoreType.DMA((2,)),
        ],
        compiler_params=pltpu.CompilerParams(
            collective_id=0,
            vmem_limit_bytes=128 * 1024 * 1024,
        ),
    )(x, router_W, route_idx, expert_W)
